# baseline (speedup 1.0000x reference)
import functools

import jax
import jax.numpy as jnp
from jax.experimental import pallas as pl
from jax.experimental.pallas import tpu as pltpu
from jax.experimental.pallas import tpu_sc as plsc

E = 64
DIM = 768
HID = 2048
TOKENS = 2048
TT = 128
GMAX = 96
PADN = GMAX * TT
SCW = 32

_VECTOR_MESH = plsc.VectorSubcoreMesh(
    core_axis_name="core", subcore_axis_name="subcore"
)


def _routing_kernel(x_ref, gw_ref, te_ref, nt_ref, pos1_ref, pos2_ref,
                    w1n_ref, w2n_ref):
    xf = x_ref[...]
    gw = gw_ref[...]
    logits = jax.lax.dot_general(
        xf, gw, (((1,), (1,)), ((), ())), preferred_element_type=jnp.float32
    )
    s = jax.nn.softmax(logits, axis=-1)
    e_iota = jax.lax.broadcasted_iota(jnp.int32, s.shape, 1)
    v1 = jnp.max(s, axis=-1)
    i1 = jnp.argmax(s, axis=-1)
    s2 = jnp.where(e_iota == i1[:, None], -jnp.inf, s)
    v2 = jnp.max(s2, axis=-1)
    i2 = jnp.argmax(s2, axis=-1)
    tot = v1 + v2 + 1e-20
    w1n_ref[...] = v1 / tot
    w2n_ref[...] = v2 / tot

    o1 = (e_iota == i1[:, None]).astype(jnp.float32)
    o2 = (e_iota == i2[:, None]).astype(jnp.float32)
    c = o1 + o2

    r_iota = jax.lax.broadcasted_iota(jnp.int32, (TOKENS, TOKENS), 0)
    c_iota = jax.lax.broadcasted_iota(jnp.int32, (TOKENS, TOKENS), 1)
    ltri = (c_iota < r_iota).astype(jnp.float32)
    excl = jax.lax.dot_general(
        ltri, c, (((1,), (0,)), ((), ())), preferred_element_type=jnp.float32
    )

    counts = jnp.sum(c, axis=0, keepdims=True)
    ptiles = jnp.floor((counts + (TT - 1)) / TT)
    ee_r = jax.lax.broadcasted_iota(jnp.int32, (E, E), 0)
    ee_c = jax.lax.broadcasted_iota(jnp.int32, (E, E), 1)
    mstrict = (ee_r < ee_c).astype(jnp.float32)
    cum_excl = jax.lax.dot_general(
        ptiles, mstrict, (((1,), (0,)), ((), ())),
        preferred_element_type=jnp.float32,
    )
    po = cum_excl * TT

    base = po + excl
    pos1 = jnp.sum(base * o1, axis=1)
    pos2 = jnp.sum(base * o2, axis=1)
    pos1_ref[...] = pos1.astype(jnp.int32)
    pos2_ref[...] = pos2.astype(jnp.int32)

    cum_incl = cum_excl + ptiles
    j_iota = jax.lax.broadcasted_iota(jnp.int32, (GMAX, E), 0)
    te = jnp.sum((cum_incl.astype(jnp.int32) <= j_iota).astype(jnp.int32),
                 axis=1)
    te_ref[...] = jnp.minimum(te, E - 1)
    nt_ref[...] = jnp.sum(ptiles, axis=1).astype(jnp.int32)


def _ffn_kernel(te_ref, nt_ref, pos1_ref, pos2_ref, w1n_ref, w2n_ref,
                xs_ref, w1_ref, w2_ref, w3_ref, ys_ref):
    j = pl.program_id(0)
    n = nt_ref[0]

    @pl.when(j < n)
    def _work():
        xs = xs_ref[...]
        w1 = w1_ref[0]
        w3 = w3_ref[0]
        w2 = w2_ref[0]
        g = jax.lax.dot_general(
            xs, w1, (((1,), (1,)), ((), ())),
            preferred_element_type=jnp.float32,
        )
        u = jax.lax.dot_general(
            xs, w3, (((1,), (1,)), ((), ())),
            preferred_element_type=jnp.float32,
        )
        h = (g * jax.nn.sigmoid(g)) * u
        o = jax.lax.dot_general(
            h, w2, (((1,), (1,)), ((), ())),
            preferred_element_type=jnp.float32,
        )
        slots = j * TT + jax.lax.broadcasted_iota(jnp.int32, (TT, 1), 0)
        hit1 = pos1_ref[...][None, :] == slots
        hit2 = pos2_ref[...][None, :] == slots
        wslot = jnp.sum(jnp.where(hit1, w1n_ref[...][None, :], 0.0) +
                        jnp.where(hit2, w2n_ref[...][None, :], 0.0),
                        axis=1, keepdims=True)
        ys_ref[...] = o * wslot


NW = 32
DISP_W = (2 * TOKENS) // NW
COMB_W = TOKENS // NW


def _sc_dispatch(xf, pos_all):
    @functools.partial(
        pl.kernel,
        out_type=jax.ShapeDtypeStruct((PADN, DIM), jnp.float32),
        mesh=_VECTOR_MESH,
        scratch_types=[
            pltpu.VMEM((1, DISP_W), jnp.int32),
            pltpu.VMEM((DISP_W, DIM), jnp.float32),
            pltpu.SemaphoreType.DMA,
        ],
    )
    def scatter_kernel(x_hbm, idx_hbm, xs_hbm, idx_v, rows_v, sem):
        wid = (jax.lax.axis_index("core") * 16
               + jax.lax.axis_index("subcore"))
        tok_base = (wid % 16) * DISP_W
        pltpu.sync_copy(idx_hbm.at[pl.ds(wid, 1)], idx_v)
        pltpu.sync_copy(x_hbm.at[pl.ds(tok_base, DISP_W)], rows_v)
        pltpu.async_copy(rows_v, xs_hbm.at[idx_v.at[0]], sem).wait()

    return scatter_kernel(xf, pos_all)


def _sc_combine(ys, pos_pair):
    half = COMB_W // 2

    @functools.partial(
        pl.kernel,
        out_type=jax.ShapeDtypeStruct((TOKENS, DIM), jnp.float32),
        mesh=_VECTOR_MESH,
        scratch_types=[
            pltpu.VMEM((1, 2 * COMB_W), jnp.int32),
            pltpu.VMEM((2 * half, DIM), jnp.float32),
            pltpu.VMEM((half, DIM), jnp.float32),
            pltpu.SemaphoreType.DMA,
        ],
    )
    def combine_kernel(ys_hbm, idx_hbm, o_hbm, idx_v, rows_v, o_v, sem):
        wid = (jax.lax.axis_index("core") * 16
               + jax.lax.axis_index("subcore"))
        base = wid * COMB_W
        pltpu.sync_copy(idx_hbm.at[pl.ds(wid, 1)], idx_v)

        @pl.loop(0, 2)
        def _half(hf):
            pltpu.async_copy(
                ys_hbm.at[idx_v.at[0, pl.ds(hf * 2 * half, 2 * half)]],
                rows_v, sem,
            ).wait()

            @pl.loop(0, half)
            def _tok(t):
                @pl.loop(0, DIM, step=16)
                def _col(c):
                    o_v.at[pl.ds(t, 1), pl.ds(c, 16)][...] = (
                        rows_v.at[pl.ds(2 * t, 1), pl.ds(c, 16)][...]
                        + rows_v.at[pl.ds(2 * t + 1, 1), pl.ds(c, 16)][...]
                    )

            pltpu.sync_copy(
                o_v, o_hbm.at[pl.ds(base + hf * half, half)]
            )

    return combine_kernel(ys, pos_pair)


@jax.jit
def kernel(x, gate_weight, w1, w2, w3):
    b, s, d = x.shape
    xf = x.reshape(-1, d)

    te, nt, pos1, pos2, w1n, w2n = pl.pallas_call(
        _routing_kernel,
        grid=(1,),
        in_specs=[
            pl.BlockSpec((TOKENS, DIM), lambda i: (0, 0)),
            pl.BlockSpec((E, DIM), lambda i: (0, 0)),
        ],
        out_specs=[
            pl.BlockSpec((GMAX,), lambda i: (0,)),
            pl.BlockSpec((1,), lambda i: (0,)),
            pl.BlockSpec((TOKENS,), lambda i: (0,)),
            pl.BlockSpec((TOKENS,), lambda i: (0,)),
            pl.BlockSpec((TOKENS,), lambda i: (0,)),
            pl.BlockSpec((TOKENS,), lambda i: (0,)),
        ],
        out_shape=[
            jax.ShapeDtypeStruct((GMAX,), jnp.int32),
            jax.ShapeDtypeStruct((1,), jnp.int32),
            jax.ShapeDtypeStruct((TOKENS,), jnp.int32),
            jax.ShapeDtypeStruct((TOKENS,), jnp.int32),
            jax.ShapeDtypeStruct((TOKENS,), jnp.float32),
            jax.ShapeDtypeStruct((TOKENS,), jnp.float32),
        ],
    )(xf, gate_weight)

    pos_all = jnp.concatenate([pos1, pos2]).reshape(NW, DISP_W)
    xs = _sc_dispatch(xf, pos_all)

    grid_spec = pltpu.PrefetchScalarGridSpec(
        num_scalar_prefetch=2,
        grid=(GMAX,),
        in_specs=[
            pl.BlockSpec((TOKENS,), lambda j, te, nt: (0,)),
            pl.BlockSpec((TOKENS,), lambda j, te, nt: (0,)),
            pl.BlockSpec((TOKENS,), lambda j, te, nt: (0,)),
            pl.BlockSpec((TOKENS,), lambda j, te, nt: (0,)),
            pl.BlockSpec((TT, DIM), lambda j, te, nt: (j, 0)),
            pl.BlockSpec((1, HID, DIM), lambda j, te, nt: (te[j], 0, 0)),
            pl.BlockSpec((1, DIM, HID), lambda j, te, nt: (te[j], 0, 0)),
            pl.BlockSpec((1, HID, DIM), lambda j, te, nt: (te[j], 0, 0)),
        ],
        out_specs=pl.BlockSpec((TT, DIM), lambda j, te, nt: (j, 0)),
    )
    ys = pl.pallas_call(
        _ffn_kernel,
        grid_spec=grid_spec,
        out_shape=jax.ShapeDtypeStruct((PADN, DIM), jnp.float32),
        compiler_params=pltpu.CompilerParams(
            dimension_semantics=("arbitrary",),
        ),
    )(te, nt, pos1, pos2, w1n, w2n, xs, w1, w2, w3)

    pos_pair = jnp.stack([pos1, pos2], axis=1).reshape(NW, 2 * COMB_W)
    out = _sc_combine(ys, pos_pair)
    return out.reshape(b, s, d)
